# baseline (speedup 1.0000x reference)
import functools

import jax
import jax.numpy as jnp
import numpy as np
from jax.experimental import pallas as pl
from jax.experimental.pallas import tpu as pltpu

_BV = 32768
_BVC = 2048
_INT_MAX = np.int32(2147483647)
_TINY = np.float32(1.1754943508222875e-38)


def _threefry_gumbel(x1_init):
    ks = (0, 42, 0 ^ 42 ^ 0x1BD11BDA)
    rots = ((13, 15, 26, 6), (17, 29, 16, 24))

    def rotl(x, r):
        return (x << np.uint32(r)) | (x >> np.uint32(32 - r))

    x1 = x1_init
    x0 = x1
    x1 = rotl(x1, 13) ^ x0
    for r in rots[0][1:]:
        x0 = x0 + x1
        x1 = rotl(x1, r)
        x1 = x1 ^ x0
    x0 = x0 + np.uint32(ks[1])
    x1 = x1 + np.uint32((ks[2] + 1) & 0xFFFFFFFF)

    for i in range(1, 5):
        for r in rots[i % 2]:
            x0 = x0 + x1
            x1 = rotl(x1, r)
            x1 = x1 ^ x0
        if ks[(i + 1) % 3]:
            x0 = x0 + np.uint32(ks[(i + 1) % 3])
        x1 = x1 + np.uint32((ks[(i + 2) % 3] + i + 1) & 0xFFFFFFFF)

    bits = x0 ^ x1
    float_bits = (bits >> jnp.uint32(9)) | jnp.uint32(0x3F800000)
    floats = jax.lax.bitcast_convert_type(float_bits, jnp.float32) - jnp.float32(1.0)
    u = jnp.maximum(floats, _TINY)
    return -jnp.log(-jnp.log(u))


def _body(t_ref, row_off_ref, xmax_ref, xsmp_ref, samp_ref,
          m_prev, mcol, pcol, icol, *, rows, vocab, nblocks, ncol):
    j = pl.program_id(0)

    @pl.when(j == 0)
    def _init():
        mcol[...] = jnp.full(mcol.shape, -jnp.inf, jnp.float32)
        pcol[...] = jnp.full(pcol.shape, -jnp.inf, jnp.float32)
        icol[...] = jnp.zeros(icol.shape, jnp.int32)

    @pl.when(j > 0)
    def _sample():
        t = t_ref[...]
        safe_t = jnp.where(t == 0, jnp.ones_like(t), t)
        gscale = jnp.where(t == 0, jnp.zeros_like(t), jnp.ones_like(t))
        m = m_prev[...] / safe_t
        base = (j - 1) * _BV

        def run(masked):
            bp = jnp.full((rows, 1), -jnp.inf, jnp.float32)
            bi = jnp.full((rows, 1), _INT_MAX, jnp.int32)
            for c in range(_BV // _BVC):
                x = xsmp_ref[:, c * _BVC:(c + 1) * _BVC]
                col = (jax.lax.broadcasted_iota(jnp.int32, x.shape, 1)
                       + (base + c * _BVC))
                if masked:
                    x = jnp.where(col < vocab, x, -jnp.inf)
                s = x / safe_t
                x1 = (col + row_off_ref[...]).astype(jnp.uint32)
                g = _threefry_gumbel(x1) * gscale
                p = (s - m) + g
                cp = jnp.max(p, axis=1, keepdims=True)
                ci = jnp.min(jnp.where(p == cp, col, _INT_MAX), axis=1,
                             keepdims=True)
                upd = cp > bp
                bi = jnp.where(upd, ci, bi)
                bp = jnp.where(upd, cp, bp)
            pcol[pl.ds(j - 1, 1)] = bp[None]
            icol[pl.ds(j - 1, 1)] = bi[None]
            mcol[pl.ds(j - 1, 1)] = m_prev[...][None]

        @pl.when(j != nblocks)
        def _full():
            run(masked=False)

        @pl.when(j == nblocks)
        def _tail():
            run(masked=True)

    @pl.when(j < nblocks)
    def _maxblock():
        x = xmax_ref[...]

        @pl.when(j != nblocks - 1)
        def _full():
            m_prev[...] = jnp.max(x, axis=1, keepdims=True)

        @pl.when(j == nblocks - 1)
        def _tail():
            col = jax.lax.broadcasted_iota(jnp.int32, x.shape, 1) + j * _BV
            xm = jnp.where(col < vocab, x, -jnp.inf)
            m_prev[...] = jnp.max(xm, axis=1, keepdims=True)

    @pl.when(j == nblocks)
    def _merge():
        t = t_ref[...]
        safe_t = jnp.where(t == 0, jnp.ones_like(t), t)
        ms = mcol[...] / safe_t
        gm = jnp.max(ms, axis=0, keepdims=True)
        shifted = pcol[...] + (ms - gm)
        best = jnp.max(shifted, axis=0, keepdims=True)
        blk_iota = jax.lax.broadcasted_iota(jnp.int32, shifted.shape, 0)
        blk = jnp.min(jnp.where(shifted == best, blk_iota, _INT_MAX), axis=0,
                      keepdims=True)
        samp_ref[...] = jnp.min(
            jnp.where(blk_iota == blk, icol[...], _INT_MAX), axis=0)


def kernel(logits, temperatures):
    rows, vocab = logits.shape
    nblocks = (vocab + _BV - 1) // _BV
    ncol = nblocks

    tcol = temperatures.astype(jnp.float32).reshape(rows, 1)
    row_off = np.arange(rows, dtype=np.int32).reshape(rows, 1) * vocab + 42

    small = pl.BlockSpec((rows, 1), lambda j: (0, 0))
    last = nblocks - 1

    samp = pl.pallas_call(
        functools.partial(_body, rows=rows, vocab=vocab, nblocks=nblocks,
                          ncol=ncol),
        grid=(nblocks + 1,),
        in_specs=[small, small,
                  pl.BlockSpec((rows, _BV),
                               lambda j: (0, jnp.minimum(j, last))),
                  pl.BlockSpec((rows, _BV),
                               lambda j: (0, jnp.maximum(j - 1, 0)))],
        out_specs=small,
        out_shape=jax.ShapeDtypeStruct((rows, 1), jnp.int32),
        scratch_shapes=[pltpu.VMEM((rows, 1), jnp.float32),
                        pltpu.VMEM((ncol, rows, 1), jnp.float32),
                        pltpu.VMEM((ncol, rows, 1), jnp.float32),
                        pltpu.VMEM((ncol, rows, 1), jnp.int32)],
        compiler_params=pltpu.CompilerParams(
            dimension_semantics=("arbitrary",)),
    )(tcol, jnp.asarray(row_off), logits, logits)

    return samp[:, 0]


# speedup vs baseline: 1.9982x; 1.9982x over previous
import functools

import jax
import jax.numpy as jnp
import numpy as np
from jax.experimental import pallas as pl
from jax.experimental.pallas import tpu as pltpu

_BV = 16384
_BVC = 2048
_INT_MAX = np.int32(2147483647)
_TINY = np.float32(1.1754943508222875e-38)


def _threefry_gumbel(x1_init):
    ks = (0, 42, 0 ^ 42 ^ 0x1BD11BDA)
    rots = ((13, 15, 26, 6), (17, 29, 16, 24))

    def rotl(x, r):
        return (x << np.uint32(r)) | (x >> np.uint32(32 - r))

    x1 = x1_init
    x0 = x1
    x1 = rotl(x1, 13) ^ x0
    for r in rots[0][1:]:
        x0 = x0 + x1
        x1 = rotl(x1, r)
        x1 = x1 ^ x0
    x0 = x0 + np.uint32(ks[1])
    x1 = x1 + np.uint32((ks[2] + 1) & 0xFFFFFFFF)

    for i in range(1, 5):
        for r in rots[i % 2]:
            x0 = x0 + x1
            x1 = rotl(x1, r)
            x1 = x1 ^ x0
        if ks[(i + 1) % 3]:
            x0 = x0 + np.uint32(ks[(i + 1) % 3])
        x1 = x1 + np.uint32((ks[(i + 2) % 3] + i + 1) & 0xFFFFFFFF)

    bits = x0 ^ x1
    float_bits = (bits >> jnp.uint32(9)) | jnp.uint32(0x3F800000)
    floats = jax.lax.bitcast_convert_type(float_bits, jnp.float32) - jnp.float32(1.0)
    u = jnp.maximum(floats, _TINY)
    return -jnp.log(-jnp.log(u))


def _body(t_ref, row_off_ref, xmax_ref, xsmp_ref, samp_ref,
          m_prev, mcol, pcol, icol, *, rows, vocab, nblocks, ncol):
    j = pl.program_id(0)

    @pl.when(j == 0)
    def _init():
        mcol[...] = jnp.full(mcol.shape, -jnp.inf, jnp.float32)
        pcol[...] = jnp.full(pcol.shape, -jnp.inf, jnp.float32)
        icol[...] = jnp.zeros(icol.shape, jnp.int32)

    @pl.when(j > 0)
    def _sample():
        t = t_ref[...]
        safe_t = jnp.where(t == 0, jnp.ones_like(t), t)
        gscale = jnp.where(t == 0, jnp.zeros_like(t), jnp.ones_like(t))
        m = m_prev[...] / safe_t
        base = (j - 1) * _BV

        def run(masked):
            bp = jnp.full((rows, 1), -jnp.inf, jnp.float32)
            bi = jnp.full((rows, 1), _INT_MAX, jnp.int32)
            for c in range(_BV // _BVC):
                x = xsmp_ref[:, c * _BVC:(c + 1) * _BVC]
                col = (jax.lax.broadcasted_iota(jnp.int32, x.shape, 1)
                       + (base + c * _BVC))
                if masked:
                    x = jnp.where(col < vocab, x, -jnp.inf)
                s = x / safe_t
                x1 = (col + row_off_ref[...]).astype(jnp.uint32)
                g = _threefry_gumbel(x1) * gscale
                p = (s - m) + g
                cp = jnp.max(p, axis=1, keepdims=True)
                ci = jnp.min(jnp.where(p == cp, col, _INT_MAX), axis=1,
                             keepdims=True)
                upd = cp > bp
                bi = jnp.where(upd, ci, bi)
                bp = jnp.where(upd, cp, bp)
            pcol[pl.ds(j - 1, 1)] = bp[None]
            icol[pl.ds(j - 1, 1)] = bi[None]
            mcol[pl.ds(j - 1, 1)] = m_prev[...][None]

        @pl.when(j != nblocks)
        def _full():
            run(masked=False)

        @pl.when(j == nblocks)
        def _tail():
            run(masked=True)

    @pl.when(j < nblocks)
    def _maxblock():
        x = xmax_ref[...]

        @pl.when(j != nblocks - 1)
        def _full():
            m_prev[...] = jnp.max(x, axis=1, keepdims=True)

        @pl.when(j == nblocks - 1)
        def _tail():
            col = jax.lax.broadcasted_iota(jnp.int32, x.shape, 1) + j * _BV
            xm = jnp.where(col < vocab, x, -jnp.inf)
            m_prev[...] = jnp.max(xm, axis=1, keepdims=True)

    @pl.when(j == nblocks)
    def _merge():
        t = t_ref[...]
        safe_t = jnp.where(t == 0, jnp.ones_like(t), t)
        ms = mcol[...] / safe_t
        gm = jnp.max(ms, axis=0, keepdims=True)
        shifted = pcol[...] + (ms - gm)
        best = jnp.max(shifted, axis=0, keepdims=True)
        blk_iota = jax.lax.broadcasted_iota(jnp.int32, shifted.shape, 0)
        blk = jnp.min(jnp.where(shifted == best, blk_iota, _INT_MAX), axis=0,
                      keepdims=True)
        samp_ref[...] = jnp.min(
            jnp.where(blk_iota == blk, icol[...], _INT_MAX), axis=0)


def kernel(logits, temperatures):
    rows, vocab = logits.shape
    nblocks = (vocab + _BV - 1) // _BV
    ncol = nblocks

    tcol = temperatures.astype(jnp.float32).reshape(rows, 1)
    row_off = np.arange(rows, dtype=np.int32).reshape(rows, 1) * vocab + 42

    small = pl.BlockSpec((rows, 1), lambda j: (0, 0))
    last = nblocks - 1

    samp = pl.pallas_call(
        functools.partial(_body, rows=rows, vocab=vocab, nblocks=nblocks,
                          ncol=ncol),
        grid=(nblocks + 1,),
        in_specs=[small, small,
                  pl.BlockSpec((rows, _BV),
                               lambda j: (0, jnp.minimum(j, last))),
                  pl.BlockSpec((rows, _BV),
                               lambda j: (0, jnp.maximum(j - 1, 0)))],
        out_specs=small,
        out_shape=jax.ShapeDtypeStruct((rows, 1), jnp.int32),
        scratch_shapes=[pltpu.VMEM((rows, 1), jnp.float32),
                        pltpu.VMEM((ncol, rows, 1), jnp.float32),
                        pltpu.VMEM((ncol, rows, 1), jnp.float32),
                        pltpu.VMEM((ncol, rows, 1), jnp.int32)],
        compiler_params=pltpu.CompilerParams(
            dimension_semantics=("arbitrary",)),
    )(tcol, jnp.asarray(row_off), logits, logits)

    return samp[:, 0]
